# Initial kernel scaffold; baseline (speedup 1.0000x reference)
import jax
import jax.numpy as jnp
from jax.experimental import pallas as pl


def kernel(utts, embedding_weight):
    raise NotImplementedError("write your pallas kernel here")


# speedup vs baseline: 1.0228x; 1.0228x over previous
import functools

import jax
import jax.numpy as jnp
from jax import lax
from jax.experimental import pallas as pl
from jax.experimental.pallas import tpu as pltpu
from jax.experimental.pallas import tpu_sc as plsc

D = 32
NC = 2
NS = 16
NW = NC * NS
CHUNK = 128


@functools.partial(jax.jit, static_argnames=("n_chunks",))
def _sc_gather(idx, table, n_chunks):
    b_per_w = n_chunks * CHUNK
    n_idx = NW * b_per_w
    mesh = plsc.VectorSubcoreMesh(core_axis_name="c", subcore_axis_name="s")

    @functools.partial(
        pl.kernel,
        mesh=mesh,
        out_type=jax.ShapeDtypeStruct((n_idx, D), jnp.float32),
        scratch_types=[
            pltpu.VMEM((n_chunks, CHUNK), jnp.int32),
            pltpu.VMEM((CHUNK, D), jnp.float32),
            pltpu.SemaphoreType.DMA,
        ],
        compiler_params=pltpu.CompilerParams(use_tc_tiling_on_sc=False),
    )
    def k(idx_hbm, table_hbm, out_hbm, idx_v, rows_v, gsem):
        wid = lax.axis_index("s") * NC + lax.axis_index("c")
        base = wid * b_per_w
        pltpu.sync_copy(idx_hbm.at[wid], idx_v)

        def body(j, carry):
            pltpu.async_copy(table_hbm.at[idx_v.at[j]], rows_v, gsem).wait()
            pltpu.sync_copy(rows_v, out_hbm.at[pl.ds(base + j * CHUNK, CHUNK)])
            return carry

        lax.fori_loop(0, n_chunks, body, 0)

    return k(idx, table)


def kernel(utts, embedding_weight):
    B, H = utts.shape
    n_idx = B * H
    n_chunks = n_idx // (NW * CHUNK)
    idx = utts.reshape(NW, n_chunks, CHUNK)
    out = _sc_gather(idx, embedding_weight, n_chunks)
    return out.reshape(B, H, D)


# speedup vs baseline: 1.1032x; 1.0786x over previous
import functools

import jax
import jax.numpy as jnp
from jax import lax
from jax.experimental import pallas as pl
from jax.experimental.pallas import tpu as pltpu
from jax.experimental.pallas import tpu_sc as plsc

D = 32
NC = 2
NS = 16
NW = NC * NS
CHUNK = 1024


@functools.partial(jax.jit, static_argnames=("n_chunks",))
def _sc_gather(idx, table, n_chunks):
    b_per_w = n_chunks * CHUNK
    n_idx = NW * b_per_w
    mesh = plsc.VectorSubcoreMesh(core_axis_name="c", subcore_axis_name="s")

    @functools.partial(
        pl.kernel,
        mesh=mesh,
        out_type=jax.ShapeDtypeStruct((n_idx, D), jnp.float32),
        scratch_types=[
            pltpu.VMEM((n_chunks, CHUNK), jnp.int32),
            pltpu.VMEM((CHUNK, D), jnp.float32),
            pltpu.SemaphoreType.DMA,
        ],
        compiler_params=pltpu.CompilerParams(use_tc_tiling_on_sc=False),
    )
    def k(idx_hbm, table_hbm, out_hbm, idx_v, rows_v, gsem):
        wid = lax.axis_index("s") * NC + lax.axis_index("c")
        base = wid * b_per_w
        pltpu.sync_copy(idx_hbm.at[wid], idx_v)

        def body(j, carry):
            pltpu.async_copy(table_hbm.at[idx_v.at[j]], rows_v, gsem).wait()
            pltpu.sync_copy(rows_v, out_hbm.at[pl.ds(base + j * CHUNK, CHUNK)])
            return carry

        lax.fori_loop(0, n_chunks, body, 0)

    return k(idx, table)


def kernel(utts, embedding_weight):
    B, H = utts.shape
    n_idx = B * H
    n_chunks = n_idx // (NW * CHUNK)
    idx = utts.reshape(NW, n_chunks, CHUNK)
    out = _sc_gather(idx, embedding_weight, n_chunks)
    return out.reshape(B, H, D)


# speedup vs baseline: 1.1080x; 1.0043x over previous
import functools

import jax
import jax.numpy as jnp
from jax import lax
from jax.experimental import pallas as pl
from jax.experimental.pallas import tpu as pltpu
from jax.experimental.pallas import tpu_sc as plsc

D = 32
NC = 2
NS = 16
NW = NC * NS
CHUNK = 1024


@functools.partial(jax.jit, static_argnames=("n_chunks",))
def _sc_gather(idx, table, n_chunks):
    b_per_w = n_chunks * CHUNK
    n_idx = NW * b_per_w
    mesh = plsc.VectorSubcoreMesh(core_axis_name="c", subcore_axis_name="s")

    @functools.partial(
        pl.kernel,
        mesh=mesh,
        out_type=jax.ShapeDtypeStruct((n_idx, D), jnp.float32),
        scratch_types=[
            pltpu.VMEM((b_per_w,), jnp.int32),
            pltpu.VMEM((CHUNK, D), jnp.float32),
            pltpu.VMEM((CHUNK, D), jnp.float32),
            pltpu.SemaphoreType.DMA,
            pltpu.SemaphoreType.DMA,
            pltpu.SemaphoreType.DMA,
            pltpu.SemaphoreType.DMA,
        ],
        compiler_params=pltpu.CompilerParams(use_tc_tiling_on_sc=False),
    )
    def k(idx_hbm, table_hbm, out_hbm, idx_v, rows0, rows1,
          gsem0, gsem1, ssem0, ssem1):
        wid = lax.axis_index("s") * NC + lax.axis_index("c")
        base = wid * b_per_w
        pltpu.sync_copy(idx_hbm.at[wid], idx_v)

        def gather(j, buf, sem):
            return pltpu.async_copy(
                table_hbm.at[idx_v.at[pl.ds(j * CHUNK, CHUNK)]], buf, sem)

        def store(j, buf, sem):
            pltpu.async_copy(
                buf, out_hbm.at[pl.ds(base + j * CHUNK, CHUNK)], sem)

        def store_wait(j, buf, sem):
            pltpu.make_async_copy(
                buf, out_hbm.at[pl.ds(base + j * CHUNK, CHUNK)], sem).wait()

        def body(t, carry):
            j0 = 2 * t
            j1 = jnp.minimum(j0 + 1, n_chunks - 1)

            @pl.when(t > 0)
            def _():
                store_wait(j0 - 2, rows0, ssem0)

            g0 = gather(j0, rows0, gsem0)

            @pl.when(t > 0)
            def _():
                store_wait(j0 - 1, rows1, ssem1)

            g1 = gather(j1, rows1, gsem1)
            g0.wait()
            store(j0, rows0, ssem0)
            g1.wait()
            store(j1, rows1, ssem1)
            return carry

        lax.fori_loop(0, (n_chunks + 1) // 2, body, 0)
        tail = n_chunks - 1
        store_wait(tail, rows0, ssem0)
        store_wait(tail, rows1, ssem1)

    return k(idx, table)


def kernel(utts, embedding_weight):
    B, H = utts.shape
    n_idx = B * H
    n_chunks = n_idx // (NW * CHUNK)
    idx = utts.reshape(NW, n_chunks * CHUNK)
    out = _sc_gather(idx, embedding_weight, n_chunks)
    return out.reshape(B, H, D)


# speedup vs baseline: 1.1102x; 1.0019x over previous
import functools

import jax
import jax.numpy as jnp
from jax import lax
from jax.experimental import pallas as pl
from jax.experimental.pallas import tpu as pltpu
from jax.experimental.pallas import tpu_sc as plsc

D = 32
NC = 2
NS = 16
NW = NC * NS
CHUNK = 640
NBUF = 4


@functools.partial(jax.jit, static_argnames=("n_chunks",))
def _sc_gather(idx, table, n_chunks):
    b_per_w = n_chunks * CHUNK
    n_idx = NW * b_per_w
    n_rounds = n_chunks // NBUF
    mesh = plsc.VectorSubcoreMesh(core_axis_name="c", subcore_axis_name="s")

    @functools.partial(
        pl.kernel,
        mesh=mesh,
        out_type=jax.ShapeDtypeStruct((n_idx, D), jnp.float32),
        scratch_types=[
            pltpu.VMEM((b_per_w,), jnp.int32),
            [pltpu.VMEM((CHUNK, D), jnp.float32) for _ in range(NBUF)],
            [pltpu.SemaphoreType.DMA for _ in range(NBUF)],
            [pltpu.SemaphoreType.DMA for _ in range(NBUF)],
        ],
        compiler_params=pltpu.CompilerParams(use_tc_tiling_on_sc=False),
    )
    def k(idx_hbm, table_hbm, out_hbm, idx_v, rows, gsems, ssems):
        wid = lax.axis_index("s") * NC + lax.axis_index("c")
        base = wid * b_per_w
        pltpu.sync_copy(idx_hbm.at[wid], idx_v)

        def gather(j, buf, sem):
            return pltpu.async_copy(
                table_hbm.at[idx_v.at[pl.ds(j * CHUNK, CHUNK)]], buf, sem)

        def store(j, buf, sem):
            pltpu.async_copy(
                buf, out_hbm.at[pl.ds(base + j * CHUNK, CHUNK)], sem)

        def store_wait(j, buf, sem):
            pltpu.make_async_copy(
                buf, out_hbm.at[pl.ds(base + j * CHUNK, CHUNK)], sem).wait()

        def body(t, carry):
            j = NBUF * t
            handles = []
            for b in range(NBUF):

                @pl.when(t > 0)
                def _(b=b):
                    store_wait(j - NBUF + b, rows[b], ssems[b])

                handles.append(gather(j + b, rows[b], gsems[b]))
            for b in range(NBUF):
                handles[b].wait()
                store(j + b, rows[b], ssems[b])
            return carry

        lax.fori_loop(0, n_rounds, body, 0)

        def drain(t, carry):
            j = NBUF * t
            for b in range(NBUF):
                store_wait(j + b, rows[b], ssems[b])
            return carry

        lax.fori_loop(n_rounds - 1, n_rounds, drain, 0)

    return k(idx, table)


def kernel(utts, embedding_weight):
    B, H = utts.shape
    n_idx = B * H
    n_chunks = n_idx // (NW * CHUNK)
    idx = utts.reshape(NW, n_chunks * CHUNK)
    out = _sc_gather(idx, embedding_weight, n_chunks)
    return out.reshape(B, H, D)


# speedup vs baseline: 1.2270x; 1.1052x over previous
import functools

import jax
import jax.numpy as jnp
from jax import lax
from jax.experimental import pallas as pl
from jax.experimental.pallas import tpu as pltpu
from jax.experimental.pallas import tpu_sc as plsc

D = 32
NC = 2
NS = 16
NW = NC * NS
CHUNK = 640
NBUF = 4


@functools.partial(jax.jit, static_argnames=("n_chunks",))
def _sc_gather(idx, table, n_chunks):
    b_per_w = n_chunks * CHUNK
    n_idx = NW * b_per_w
    n_rounds = n_chunks // NBUF
    mesh = plsc.VectorSubcoreMesh(core_axis_name="c", subcore_axis_name="s")

    @functools.partial(
        pl.kernel,
        mesh=mesh,
        out_type=jax.ShapeDtypeStruct((n_idx, D), jnp.float32),
        scratch_types=[
            pltpu.VMEM((b_per_w,), jnp.int32),
            [pltpu.VMEM((CHUNK, D), jnp.float32) for _ in range(NBUF)],
            [pltpu.SemaphoreType.DMA for _ in range(NBUF)],
            [pltpu.SemaphoreType.DMA for _ in range(NBUF)],
        ],
        compiler_params=pltpu.CompilerParams(use_tc_tiling_on_sc=False),
    )
    def k(idx_hbm, table_hbm, out_hbm, idx_v, rows, gsems, ssems):
        wid = lax.axis_index("s") * NC + lax.axis_index("c")
        base = wid * b_per_w
        pltpu.sync_copy(idx_hbm.at[wid], idx_v)

        def gather(j, buf, sem):
            return pltpu.async_copy(
                table_hbm.at[idx_v.at[pl.ds(j * CHUNK, CHUNK)]], buf, sem)

        def store(j, buf, sem):
            pltpu.async_copy(
                buf, out_hbm.at[pl.ds(base + j * CHUNK, CHUNK)], sem)

        def store_wait(j, buf, sem):
            pltpu.make_async_copy(
                buf, out_hbm.at[pl.ds(base + j * CHUNK, CHUNK)], sem).wait()

        def body(t, carry):
            j = NBUF * t
            handles = []
            for b in range(NBUF):

                @pl.when(t > 0)
                def _(b=b):
                    store_wait(j - NBUF + b, rows[b], ssems[b])

                handles.append(gather(j + b, rows[b], gsems[b]))
            for b in range(NBUF):
                handles[b].wait()
                store(j + b, rows[b], ssems[b])
            return carry

        lax.fori_loop(0, n_rounds, body, 0)

        def drain(t, carry):
            j = NBUF * t
            for b in range(NBUF):
                store_wait(j + b, rows[b], ssems[b])
            return carry

        lax.fori_loop(n_rounds - 1, n_rounds, drain, 0)

    return k(idx, table)


def kernel(utts, embedding_weight):
    B, H = utts.shape
    half = B // 2
    n_chunks = half * H // (NW * CHUNK)
    outs = []
    for part in (utts[:half], utts[half:]):
        idx = part.reshape(NW, n_chunks * CHUNK)
        out = _sc_gather(idx, embedding_weight, n_chunks)
        outs.append(out.reshape(half, H, D))
    return jnp.concatenate(outs, axis=0)


# speedup vs baseline: 1.2546x; 1.0225x over previous
import functools

import jax
import jax.numpy as jnp
from jax import lax
from jax.experimental import pallas as pl
from jax.experimental.pallas import tpu as pltpu
from jax.experimental.pallas import tpu_sc as plsc

D = 32
NC = 2
NS = 16
NW = NC * NS
CHUNK = 640
NBUF = 2
NSPLIT = 4


@functools.partial(jax.jit, static_argnames=("n_chunks",))
def _sc_gather(idx, table, n_chunks):
    b_per_w = n_chunks * CHUNK
    n_idx = NW * b_per_w
    n_rounds = n_chunks // NBUF
    mesh = plsc.VectorSubcoreMesh(core_axis_name="c", subcore_axis_name="s")

    @functools.partial(
        pl.kernel,
        mesh=mesh,
        out_type=jax.ShapeDtypeStruct((n_idx, D), jnp.float32),
        scratch_types=[
            pltpu.VMEM((b_per_w,), jnp.int32),
            [pltpu.VMEM((CHUNK, D), jnp.float32) for _ in range(NBUF)],
            [pltpu.SemaphoreType.DMA for _ in range(NBUF)],
            [pltpu.SemaphoreType.DMA for _ in range(NBUF)],
        ],
        compiler_params=pltpu.CompilerParams(use_tc_tiling_on_sc=False),
    )
    def k(idx_hbm, table_hbm, out_hbm, idx_v, rows, gsems, ssems):
        wid = lax.axis_index("s") * NC + lax.axis_index("c")
        base = wid * b_per_w
        pltpu.sync_copy(idx_hbm.at[wid], idx_v)

        def gather(j, buf, sem):
            return pltpu.async_copy(
                table_hbm.at[idx_v.at[pl.ds(j * CHUNK, CHUNK)]], buf, sem)

        def store(j, buf, sem):
            pltpu.async_copy(
                buf, out_hbm.at[pl.ds(base + j * CHUNK, CHUNK)], sem)

        def store_wait(j, buf, sem):
            pltpu.make_async_copy(
                buf, out_hbm.at[pl.ds(base + j * CHUNK, CHUNK)], sem).wait()

        def body(t, carry):
            j = NBUF * t
            handles = []
            for b in range(NBUF):

                @pl.when(t > 0)
                def _(b=b):
                    store_wait(j - NBUF + b, rows[b], ssems[b])

                handles.append(gather(j + b, rows[b], gsems[b]))
            for b in range(NBUF):
                handles[b].wait()
                store(j + b, rows[b], ssems[b])
            return carry

        lax.fori_loop(0, n_rounds, body, 0)

        def drain(t, carry):
            j = NBUF * t
            for b in range(NBUF):
                store_wait(j + b, rows[b], ssems[b])
            return carry

        lax.fori_loop(n_rounds - 1, n_rounds, drain, 0)

    return k(idx, table)


def kernel(utts, embedding_weight):
    B, H = utts.shape
    part_rows = B // NSPLIT
    n_chunks = part_rows * H // (NW * CHUNK)
    outs = []
    for p in range(NSPLIT):
        part = utts[p * part_rows:(p + 1) * part_rows]
        idx = part.reshape(NW, n_chunks * CHUNK)
        out = _sc_gather(idx, embedding_weight, n_chunks)
        outs.append(out.reshape(part_rows, H, D))
    return jnp.concatenate(outs, axis=0)
